# baseline (speedup 1.0000x reference)
import jax
import jax.numpy as jnp
from jax import lax
from jax.experimental import pallas as pl
from jax.experimental.pallas import tpu as pltpu
from jax.experimental.pallas import tpu_sc as plsc

L = 16
NC, NS = 2, 16
NW = NC * NS
RB = 128
PC = 4
ROW = RB * PC
USED = RB * 3
CHUNK_ROWS = 32
CHUNK_RAYS = CHUNK_ROWS * RB
NCHUNKS = 8


def _take(vec, idx):
    return jnp.take_along_axis(vec, idx, axis=0, mode="promise_in_bounds")


def _body(rgb_hbm, calt_hbm, out_hbm, calt_v,
          rgb_v0, rgb_v1, rgb_v2, rgb_v3, rgb_v4, rgb_v5,
          sem_in0, sem_in1, sem_in2, sem_in3, sem_in4, sem_in5,
          sem_out0, sem_out1, sem_out2, sem_out3, sem_out4, sem_out5):
    rgb_bufs = (rgb_v0, rgb_v1, rgb_v2, rgb_v3, rgb_v4, rgb_v5)
    sems_in = (sem_in0, sem_in1, sem_in2, sem_in3, sem_in4, sem_in5)
    sems_out = (sem_out0, sem_out1, sem_out2, sem_out3, sem_out4, sem_out5)

    wid = lax.axis_index("s") * NC + lax.axis_index("c")
    rows_per_w = out_hbm.shape[0] // NW

    pltpu.sync_copy(calt_hbm, calt_v)
    lane = lax.iota(jnp.int32, L)
    cam0 = lane == 0
    ts = [jnp.where(cam0, 1.0, calt_v[pl.ds(16 * c, L)]) for c in range(3)]
    to = [jnp.where(cam0, 0.0, calt_v[pl.ds(16 * c, L)]) for c in range(3, 6)]

    def start_in(k, slot):
        row0 = wid * rows_per_w + k * CHUNK_ROWS
        return (
            pltpu.async_copy(
                rgb_hbm.at[pl.ds(row0, CHUNK_ROWS)],
                rgb_bufs[slot], sems_in[slot]
            ),
        )

    def start_out(k, slot):
        row0 = wid * rows_per_w + k * CHUNK_ROWS
        return pltpu.async_copy(
            rgb_bufs[slot].at[:, pl.ds(0, 3), :],
            out_hbm.at[pl.ds(row0, CHUNK_ROWS), pl.ds(0, 3), :],
            sems_out[slot]
        )

    def compute(slot):
        rgb_v = rgb_bufs[slot]

        def do_row(g, carry):
            for w in range(RB // L):
                cam = rgb_v[g, 3, pl.ds(L * w, L)].astype(jnp.int32)
                for c in range(3):
                    x = rgb_v[g, c, pl.ds(L * w, L)]
                    rgb_v[g, c, pl.ds(L * w, L)] = (
                        x * _take(ts[c], cam) + _take(to[c], cam)
                    )
            return carry

        lax.fori_loop(0, CHUNK_ROWS, do_row, 0)

    in_descs = {}
    out_descs = {}
    for j in range(3):
        in_descs[j] = start_in(j, j)
    for k in range(NCHUNKS):
        slot = k % 6
        if k + 3 < NCHUNKS:
            if k >= 3:
                out_descs.pop(k - 3).wait()
            in_descs[k + 3] = start_in(k + 3, (k + 3) % 6)
        for d in in_descs.pop(k):
            d.wait()
        compute(slot)
        out_descs[k] = start_out(k, slot)
    for k in sorted(out_descs):
        out_descs.pop(k).wait()


def kernel(rgb_map, real_cam_idx, cal):
    b = rgb_map.shape[0]
    nrows = b // RB
    assert b == NW * NCHUNKS * CHUNK_ROWS * RB, b
    cam_f32 = real_cam_idx.astype(jnp.float32)
    rgb4 = jnp.concatenate([rgb_map, cam_f32[:, None]], axis=1)
    rgb_rows = jnp.transpose(rgb4.reshape(nrows, RB, PC), (0, 2, 1))

    mesh = plsc.VectorSubcoreMesh(
        core_axis_name="c", subcore_axis_name="s", num_cores=NC, num_subcores=NS
    )
    run = pl.kernel(
        _body,
        out_type=jax.ShapeDtypeStruct((nrows, PC, RB), jnp.float32),
        mesh=mesh,
        scratch_types=[
            pltpu.VMEM((96,), jnp.float32),
            pltpu.VMEM((CHUNK_ROWS, PC, RB), jnp.float32),
            pltpu.VMEM((CHUNK_ROWS, PC, RB), jnp.float32),
            pltpu.VMEM((CHUNK_ROWS, PC, RB), jnp.float32),
            pltpu.VMEM((CHUNK_ROWS, PC, RB), jnp.float32),
            pltpu.VMEM((CHUNK_ROWS, PC, RB), jnp.float32),
            pltpu.VMEM((CHUNK_ROWS, PC, RB), jnp.float32),
            pltpu.SemaphoreType.DMA,
            pltpu.SemaphoreType.DMA,
            pltpu.SemaphoreType.DMA,
            pltpu.SemaphoreType.DMA,
            pltpu.SemaphoreType.DMA,
            pltpu.SemaphoreType.DMA,
            pltpu.SemaphoreType.DMA,
            pltpu.SemaphoreType.DMA,
            pltpu.SemaphoreType.DMA,
            pltpu.SemaphoreType.DMA,
            pltpu.SemaphoreType.DMA,
            pltpu.SemaphoreType.DMA,
        ],
    )
    out_rows = run(rgb_rows, cal.T.reshape(-1))
    out4 = jnp.transpose(out_rows, (0, 2, 1)).reshape(b, PC)
    return out4[:, :3]


# speedup vs baseline: 1.0017x; 1.0017x over previous
import jax
import jax.numpy as jnp
from jax import lax
from jax.experimental import pallas as pl
from jax.experimental.pallas import tpu as pltpu
from jax.experimental.pallas import tpu_sc as plsc

L = 16
NC, NS = 2, 16
NW = NC * NS
RB = 128
PC = 4
CHUNK_ROWS = 32
NCHUNKS = 8


def _take(vec, idx):
    return jnp.take_along_axis(vec, idx, axis=0, mode="promise_in_bounds")


def _body(rgb_hbm, calt_hbm, out_hbm, calt_v,
          rgb_v0, rgb_v1, rgb_v2, rgb_v3, rgb_v4, rgb_v5,
          sem_in0, sem_in1, sem_in2, sem_in3, sem_in4, sem_in5,
          sem_out0, sem_out1, sem_out2, sem_out3, sem_out4, sem_out5):
    rgb_bufs = (rgb_v0, rgb_v1, rgb_v2, rgb_v3, rgb_v4, rgb_v5)
    sems_in = (sem_in0, sem_in1, sem_in2, sem_in3, sem_in4, sem_in5)
    sems_out = (sem_out0, sem_out1, sem_out2, sem_out3, sem_out4, sem_out5)

    wid = lax.axis_index("s") * NC + lax.axis_index("c")
    rows_per_w = out_hbm.shape[0] // NW

    pltpu.sync_copy(calt_hbm, calt_v)
    lane = lax.iota(jnp.int32, L)
    cam0 = lane == 0
    ts = [jnp.where(cam0, 1.0, calt_v[pl.ds(16 * c, L)]) for c in range(3)]
    to = [jnp.where(cam0, 0.0, calt_v[pl.ds(16 * c, L)]) for c in range(3, 6)]

    def start_in(k, slot):
        row0 = wid * rows_per_w + k * CHUNK_ROWS
        return (
            pltpu.async_copy(
                rgb_hbm.at[pl.ds(row0, CHUNK_ROWS)],
                rgb_bufs[slot], sems_in[slot]
            ),
        )

    def start_out(k, slot):
        row0 = wid * rows_per_w + k * CHUNK_ROWS
        return pltpu.async_copy(
            rgb_bufs[slot].at[:, pl.ds(0, 3), :],
            out_hbm.at[pl.ds(row0, CHUNK_ROWS), pl.ds(0, 3), :],
            sems_out[slot]
        )

    def compute(slot):
        rgb_v = rgb_bufs[slot]

        def do_row(g, carry):
            for w in range(RB // L):
                cam = rgb_v[g, 3, pl.ds(L * w, L)].astype(jnp.int32)
                for c in range(3):
                    x = rgb_v[g, c, pl.ds(L * w, L)]
                    rgb_v[g, c, pl.ds(L * w, L)] = (
                        x * _take(ts[c], cam) + _take(to[c], cam)
                    )
            return carry

        lax.fori_loop(0, CHUNK_ROWS, do_row, 0)

    in_descs = {}
    out_descs = {}
    for j in range(3):
        in_descs[j] = start_in(j, j)
    for k in range(NCHUNKS):
        slot = k % 6
        if k + 3 < NCHUNKS:
            if k >= 3:
                out_descs.pop(k - 3).wait()
            in_descs[k + 3] = start_in(k + 3, (k + 3) % 6)
        for d in in_descs.pop(k):
            d.wait()
        compute(slot)
        out_descs[k] = start_out(k, slot)
    for k in sorted(out_descs):
        out_descs.pop(k).wait()


def kernel(rgb_map, real_cam_idx, cal):
    b = rgb_map.shape[0]
    nrows = b // RB
    assert b == NW * NCHUNKS * CHUNK_ROWS * RB, b
    cam_f32 = real_cam_idx.astype(jnp.float32)
    rgb4 = jnp.concatenate([rgb_map, cam_f32[:, None]], axis=1)
    rgb_rows = jnp.transpose(rgb4.reshape(nrows, RB, PC), (0, 2, 1))

    mesh = plsc.VectorSubcoreMesh(
        core_axis_name="c", subcore_axis_name="s", num_cores=NC, num_subcores=NS
    )
    run = pl.kernel(
        _body,
        out_type=jax.ShapeDtypeStruct((nrows, PC, RB), jnp.float32),
        mesh=mesh,
        scratch_types=[
            pltpu.VMEM((96,), jnp.float32),
            pltpu.VMEM((CHUNK_ROWS, PC, RB), jnp.float32),
            pltpu.VMEM((CHUNK_ROWS, PC, RB), jnp.float32),
            pltpu.VMEM((CHUNK_ROWS, PC, RB), jnp.float32),
            pltpu.VMEM((CHUNK_ROWS, PC, RB), jnp.float32),
            pltpu.VMEM((CHUNK_ROWS, PC, RB), jnp.float32),
            pltpu.VMEM((CHUNK_ROWS, PC, RB), jnp.float32),
            pltpu.SemaphoreType.DMA,
            pltpu.SemaphoreType.DMA,
            pltpu.SemaphoreType.DMA,
            pltpu.SemaphoreType.DMA,
            pltpu.SemaphoreType.DMA,
            pltpu.SemaphoreType.DMA,
            pltpu.SemaphoreType.DMA,
            pltpu.SemaphoreType.DMA,
            pltpu.SemaphoreType.DMA,
            pltpu.SemaphoreType.DMA,
            pltpu.SemaphoreType.DMA,
            pltpu.SemaphoreType.DMA,
        ],
    )
    out_rows = run(rgb_rows, cal.T.reshape(-1))
    out4 = jnp.transpose(out_rows, (0, 2, 1)).reshape(b, PC)
    return out4[:, :3]
